# baseline (speedup 1.0000x reference)
import functools

import jax
import jax.numpy as jnp
from jax import lax
from jax.experimental import pallas as pl
from jax.experimental.pallas import tpu as pltpu
from jax.experimental.pallas import tpu_sc as plsc

_TB = 2048
_E = 16
_L = 16
_NW = 32
_NEG_INF = float("-inf")


def _mm_body(w_ref, hs_ref, out_ref):
    out_ref[...] = jax.lax.dot_general(
        w_ref[...], hs_ref[...], (((1,), (1,)), ((), ())),
        preferred_element_type=jnp.float32,
    )


def _logits_T(hs, W):
    T, d = hs.shape
    return pl.pallas_call(
        _mm_body,
        grid=(T // _TB,),
        in_specs=[
            pl.BlockSpec((_E, d), lambda i: (0, 0)),
            pl.BlockSpec((_TB, d), lambda i: (i, 0)),
        ],
        out_specs=pl.BlockSpec((_E, _TB), lambda i: (0, i)),
        out_shape=jax.ShapeDtypeStruct((_E, T), jnp.float32),
    )(W, hs)


def _top2_rows(lg):
    row = lax.broadcasted_iota(jnp.int32, lg.shape, 0)
    m1 = jnp.max(lg, axis=0, keepdims=True)
    i1 = jnp.min(jnp.where(lg == m1, row, _E), axis=0, keepdims=True)
    masked = jnp.where(row == i1, _NEG_INF, lg)
    m2 = jnp.max(masked, axis=0, keepdims=True)
    i2 = jnp.min(jnp.where(masked == m2, row, _E), axis=0, keepdims=True)
    t = jnp.exp(m2 - m1)
    denom = 1.0 + t
    return 1.0 / denom, t / denom, i1, i2


def _mm_route_body(w_ref, hs_ref, wout_ref, eout_ref):
    lg = jax.lax.dot_general(
        w_ref[...], hs_ref[...], (((1,), (1,)), ((), ())),
        preferred_element_type=jnp.float32,
    )
    w1, w2, i1, i2 = _top2_rows(lg)
    wout_ref[...] = jnp.concatenate([w1, w2], axis=0)
    eout_ref[...] = jnp.concatenate([i1, i2], axis=0)


def _route_tc(hs, W):
    T, d = hs.shape
    return pl.pallas_call(
        _mm_route_body,
        grid=(T // _TB,),
        in_specs=[
            pl.BlockSpec((_E, d), lambda i: (0, 0)),
            pl.BlockSpec((_TB, d), lambda i: (i, 0)),
        ],
        out_specs=(
            pl.BlockSpec((2, _TB), lambda i: (0, i)),
            pl.BlockSpec((2, _TB), lambda i: (0, i)),
        ),
        out_shape=(
            jax.ShapeDtypeStruct((2, T), jnp.float32),
            jax.ShapeDtypeStruct((2, T), jnp.int32),
        ),
    )(W, hs)


def _route_sc_body(lg_hbm, w_hbm, e_hbm, blk, w1b, w2b, e1b, e2b):
    cpt = lax.axis_index("s") * 2 + lax.axis_index("c")
    chunk = blk.shape[1]
    base = cpt * chunk
    pltpu.sync_copy(lg_hbm.at[:, pl.ds(base, chunk)], blk)

    def group(g, carry):
        t0 = g * _L
        max1 = blk[0, pl.ds(t0, _L)]
        idx1 = jnp.zeros((_L,), jnp.int32)
        max2 = jnp.full((_L,), _NEG_INF, jnp.float32)
        idx2 = jnp.zeros((_L,), jnp.int32)
        for e in range(1, _E):
            v = blk[e, pl.ds(t0, _L)]
            ev = jnp.full((_L,), e, jnp.int32)
            gt1 = v > max1
            gt2 = v > max2
            max2n = jnp.where(gt1, max1, jnp.where(gt2, v, max2))
            idx2n = jnp.where(gt1, idx1, jnp.where(gt2, ev, idx2))
            max1 = jnp.where(gt1, v, max1)
            idx1 = jnp.where(gt1, ev, idx1)
            max2, idx2 = max2n, idx2n
        t = jnp.exp(max2 - max1)
        denom = 1.0 + t
        sl = pl.ds(t0, _L)
        w1b[sl] = 1.0 / denom
        w2b[sl] = t / denom
        e1b[sl] = idx1
        e2b[sl] = idx2
        return carry

    lax.fori_loop(0, chunk // _L, group, 0)
    pltpu.sync_copy(w1b, w_hbm.at[0, pl.ds(base, chunk)])
    pltpu.sync_copy(w2b, w_hbm.at[1, pl.ds(base, chunk)])
    pltpu.sync_copy(e1b, e_hbm.at[0, pl.ds(base, chunk)])
    pltpu.sync_copy(e2b, e_hbm.at[1, pl.ds(base, chunk)])


def _route_sc(logits_T):
    E, T = logits_T.shape
    chunk = T // _NW
    mesh = plsc.VectorSubcoreMesh(core_axis_name="c", subcore_axis_name="s")
    fn = functools.partial(
        pl.kernel,
        mesh=mesh,
        out_type=(
            jax.ShapeDtypeStruct((2, T), jnp.float32),
            jax.ShapeDtypeStruct((2, T), jnp.int32),
        ),
        scratch_types=[
            pltpu.VMEM((E, chunk), jnp.float32),
            pltpu.VMEM((chunk,), jnp.float32),
            pltpu.VMEM((chunk,), jnp.float32),
            pltpu.VMEM((chunk,), jnp.int32),
            pltpu.VMEM((chunk,), jnp.int32),
        ],
    )(_route_sc_body)
    return fn(logits_T)


@jax.jit
def kernel(hidden_states, W):
    hs = hidden_states.reshape(-1, hidden_states.shape[-1])
    T = hs.shape[0]
    T0 = T // 2
    lt0 = _logits_T(hs[:T0], W)
    w_h0, e_h0 = _route_sc(lt0)
    w_h1, e_h1 = _route_tc(hs[T0:], W)
    top_weights = jnp.concatenate([w_h0, w_h1], axis=1).T
    top_experts = jnp.concatenate([e_h0, e_h1], axis=1).T
    return (top_weights, top_experts)


# speedup vs baseline: 2.2740x; 2.2740x over previous
import functools

import jax
import jax.numpy as jnp
from jax import lax
from jax.experimental import pallas as pl
from jax.experimental.pallas import tpu as pltpu
from jax.experimental.pallas import tpu_sc as plsc

_TB = 2048
_E = 16
_L = 16
_NW = 32
_NEG_INF = float("-inf")


def _mm_body(w_ref, hs_ref, out_ref):
    out_ref[...] = jax.lax.dot_general(
        w_ref[...], hs_ref[...], (((1,), (1,)), ((), ())),
        preferred_element_type=jnp.float32,
    )


def _logits_T(hs, W, off, nblk):
    T, d = hs.shape
    return pl.pallas_call(
        _mm_body,
        grid=(nblk,),
        in_specs=[
            pl.BlockSpec((_E, d), lambda i: (0, 0)),
            pl.BlockSpec((_TB, d), lambda i: (i + off, 0)),
        ],
        out_specs=pl.BlockSpec((_E, _TB), lambda i: (0, i)),
        out_shape=jax.ShapeDtypeStruct((_E, nblk * _TB), jnp.float32),
    )(W, hs)


def _top2_rows(lg):
    row = lax.broadcasted_iota(jnp.int32, lg.shape, 0)
    m1 = jnp.max(lg, axis=0, keepdims=True)
    i1 = jnp.min(jnp.where(lg == m1, row, _E), axis=0, keepdims=True)
    masked = jnp.where(row == i1, _NEG_INF, lg)
    m2 = jnp.max(masked, axis=0, keepdims=True)
    i2 = jnp.min(jnp.where(masked == m2, row, _E), axis=0, keepdims=True)
    t = jnp.exp(m2 - m1)
    denom = 1.0 + t
    return 1.0 / denom, t / denom, i1, i2


def _mm_route_body(w_ref, hs_ref, wout_ref, eout_ref):
    lg = jax.lax.dot_general(
        w_ref[...], hs_ref[...], (((1,), (1,)), ((), ())),
        preferred_element_type=jnp.float32,
    )
    w1, w2, i1, i2 = _top2_rows(lg)
    wout_ref[...] = jnp.concatenate([w1, w2], axis=0)
    eout_ref[...] = jnp.concatenate([i1, i2], axis=0)


def _route_tc(hs, W, off, nblk):
    T, d = hs.shape
    return pl.pallas_call(
        _mm_route_body,
        grid=(nblk,),
        in_specs=[
            pl.BlockSpec((_E, d), lambda i: (0, 0)),
            pl.BlockSpec((_TB, d), lambda i: (i + off, 0)),
        ],
        out_specs=(
            pl.BlockSpec((2, _TB), lambda i: (0, i)),
            pl.BlockSpec((2, _TB), lambda i: (0, i)),
        ),
        out_shape=(
            jax.ShapeDtypeStruct((2, nblk * _TB), jnp.float32),
            jax.ShapeDtypeStruct((2, nblk * _TB), jnp.int32),
        ),
    )(W, hs)


def _route_sc_body(lg_hbm, w_hbm, e_hbm, blk, w1b, w2b, e1b, e2b):
    cpt = lax.axis_index("s") * 2 + lax.axis_index("c")
    chunk = blk.shape[1]
    base = cpt * chunk
    pltpu.sync_copy(lg_hbm.at[:, pl.ds(base, chunk)], blk)

    def group(g, carry):
        t0 = g * _L
        max1 = blk[0, pl.ds(t0, _L)]
        idx1 = jnp.zeros((_L,), jnp.int32)
        max2 = jnp.full((_L,), _NEG_INF, jnp.float32)
        idx2 = jnp.zeros((_L,), jnp.int32)
        for e in range(1, _E):
            v = blk[e, pl.ds(t0, _L)]
            ev = jnp.full((_L,), e, jnp.int32)
            gt1 = v > max1
            gt2 = v > max2
            max2n = jnp.where(gt1, max1, jnp.where(gt2, v, max2))
            idx2n = jnp.where(gt1, idx1, jnp.where(gt2, ev, idx2))
            max1 = jnp.where(gt1, v, max1)
            idx1 = jnp.where(gt1, ev, idx1)
            max2, idx2 = max2n, idx2n
        t = jnp.exp(max2 - max1)
        denom = 1.0 + t
        sl = pl.ds(t0, _L)
        w1b[sl] = 1.0 / denom
        w2b[sl] = t / denom
        e1b[sl] = idx1
        e2b[sl] = idx2
        return carry

    lax.fori_loop(0, chunk // _L, group, 0)
    pltpu.sync_copy(w1b, w_hbm.at[0, pl.ds(base, chunk)])
    pltpu.sync_copy(w2b, w_hbm.at[1, pl.ds(base, chunk)])
    pltpu.sync_copy(e1b, e_hbm.at[0, pl.ds(base, chunk)])
    pltpu.sync_copy(e2b, e_hbm.at[1, pl.ds(base, chunk)])


def _route_sc(logits_T):
    E, T = logits_T.shape
    chunk = T // _NW
    mesh = plsc.VectorSubcoreMesh(core_axis_name="c", subcore_axis_name="s")
    fn = functools.partial(
        pl.kernel,
        mesh=mesh,
        out_type=(
            jax.ShapeDtypeStruct((2, T), jnp.float32),
            jax.ShapeDtypeStruct((2, T), jnp.int32),
        ),
        scratch_types=[
            pltpu.VMEM((E, chunk), jnp.float32),
            pltpu.VMEM((chunk,), jnp.float32),
            pltpu.VMEM((chunk,), jnp.float32),
            pltpu.VMEM((chunk,), jnp.int32),
            pltpu.VMEM((chunk,), jnp.int32),
        ],
    )(_route_sc_body)
    return fn(logits_T)


@jax.jit
def kernel(hidden_states, W):
    hs = hidden_states.reshape(-1, hidden_states.shape[-1])
    T = hs.shape[0]
    nb = T // _TB
    nb0 = nb // 2
    lt0 = _logits_T(hs, W, 0, nb0)
    w_h0, e_h0 = _route_sc(lt0)
    w_h1, e_h1 = _route_tc(hs, W, nb0, nb - nb0)
    top_weights = jnp.concatenate([w_h0, w_h1], axis=1).T
    top_experts = jnp.concatenate([e_h0, e_h1], axis=1).T
    return (top_weights, top_experts)


# speedup vs baseline: 3.3860x; 1.4890x over previous
import jax
import jax.numpy as jnp
from jax import lax
from jax.experimental import pallas as pl

_TB = 2048
_E = 16
_NEG_INF = float("-inf")


def _body(w_ref, hs_ref, wout_ref, eout_ref):
    lg = jax.lax.dot_general(
        w_ref[...], hs_ref[...], (((1,), (1,)), ((), ())),
        preferred_element_type=jnp.float32,
    )
    row = lax.broadcasted_iota(jnp.int32, lg.shape, 0)
    m1 = jnp.max(lg, axis=0, keepdims=True)
    i1 = jnp.min(jnp.where(lg == m1, row, _E), axis=0, keepdims=True)
    masked = jnp.where(row == i1, _NEG_INF, lg)
    m2 = jnp.max(masked, axis=0, keepdims=True)
    i2 = jnp.min(jnp.where(masked == m2, row, _E), axis=0, keepdims=True)
    t = jnp.exp(m2 - m1)
    denom = 1.0 + t
    wout_ref[...] = jnp.concatenate([1.0 / denom, t / denom], axis=0)
    eout_ref[...] = jnp.concatenate([i1, i2], axis=0)


@jax.jit
def kernel(hidden_states, W):
    hs = hidden_states.reshape(-1, hidden_states.shape[-1])
    T, d = hs.shape
    w2t, e2t = pl.pallas_call(
        _body,
        grid=(T // _TB,),
        in_specs=[
            pl.BlockSpec((_E, d), lambda i: (0, 0)),
            pl.BlockSpec((_TB, d), lambda i: (i, 0)),
        ],
        out_specs=(
            pl.BlockSpec((2, _TB), lambda i: (0, i)),
            pl.BlockSpec((2, _TB), lambda i: (0, i)),
        ),
        out_shape=(
            jax.ShapeDtypeStruct((2, T), jnp.float32),
            jax.ShapeDtypeStruct((2, T), jnp.int32),
        ),
    )(W, hs)
    return (w2t.T, e2t.T)


# speedup vs baseline: 3.5616x; 1.0519x over previous
import jax
import jax.numpy as jnp
from jax import lax
from jax.experimental import pallas as pl

_TB = 1024
_E = 16
_NEG_INF = float("-inf")


def _body(w_ref, hs_ref, wout_ref, eout_ref):
    lg = jax.lax.dot_general(
        w_ref[...], hs_ref[...], (((1,), (1,)), ((), ())),
        preferred_element_type=jnp.float32,
    )
    row = lax.broadcasted_iota(jnp.int32, lg.shape, 0)
    m1 = jnp.max(lg, axis=0, keepdims=True)
    i1 = jnp.min(jnp.where(lg == m1, row, _E), axis=0, keepdims=True)
    masked = jnp.where(row == i1, _NEG_INF, lg)
    m2 = jnp.max(masked, axis=0, keepdims=True)
    i2 = jnp.min(jnp.where(masked == m2, row, _E), axis=0, keepdims=True)
    t = jnp.exp(m2 - m1)
    denom = 1.0 + t
    wout_ref[...] = jnp.concatenate([1.0 / denom, t / denom], axis=0)
    eout_ref[...] = jnp.concatenate([i1, i2], axis=0)


@jax.jit
def kernel(hidden_states, W):
    hs = hidden_states.reshape(-1, hidden_states.shape[-1])
    T, d = hs.shape
    w2t, e2t = pl.pallas_call(
        _body,
        grid=(T // _TB,),
        in_specs=[
            pl.BlockSpec((_E, d), lambda i: (0, 0)),
            pl.BlockSpec((_TB, d), lambda i: (i, 0)),
        ],
        out_specs=(
            pl.BlockSpec((2, _TB), lambda i: (0, i)),
            pl.BlockSpec((2, _TB), lambda i: (0, i)),
        ),
        out_shape=(
            jax.ShapeDtypeStruct((2, T), jnp.float32),
            jax.ShapeDtypeStruct((2, T), jnp.int32),
        ),
    )(W, hs)
    return (w2t.T, e2t.T)
